# baseline (speedup 1.0000x reference)
import functools

import jax
import jax.numpy as jnp
from jax import lax
from jax.experimental import pallas as pl
from jax.experimental.pallas import tpu as pltpu
from jax.experimental.pallas import tpu_sc as plsc

N_CORES = 2
N_SUBCORES = 16
N_TILES = N_CORES * N_SUBCORES
K = 128
ZR = 128


@functools.partial(jax.jit, static_argnames=("n", "b", "per_tile"))
def _sc_spmm(xT, rows, cols, vals, *, n, b, per_tile):
    chunks = per_tile // K
    rows_per_tile = n // N_SUBCORES
    mesh = plsc.VectorSubcoreMesh(core_axis_name="c", subcore_axis_name="s")

    @functools.partial(
        pl.kernel,
        out_type=(
            jax.ShapeDtypeStruct((n, b), jnp.float32),
            jax.ShapeDtypeStruct((n, b), jnp.float32),
        ),
        mesh=mesh,
        compiler_params=pltpu.CompilerParams(use_tc_tiling_on_sc=False),
        scratch_types=[
            pltpu.VMEM((4, K), jnp.int32),
            pltpu.VMEM((4, K), jnp.int32),
            pltpu.VMEM((4, K), jnp.float32),
            pltpu.VMEM((2, K, b), jnp.float32),
            pltpu.VMEM((2, K, b), jnp.float32),
            pltpu.VMEM((ZR, b), jnp.float32),
            pltpu.VMEM_SHARED((n, b), jnp.float32),
            pltpu.SemaphoreType.DMA,
            pltpu.SemaphoreType.DMA,
            pltpu.SemaphoreType.DMA,
            pltpu.SemaphoreType.DMA,
            pltpu.SemaphoreType.DMA,
            pltpu.SemaphoreType.DMA,
            pltpu.SemaphoreType.DMA,
            pltpu.SemaphoreType.DMA,
        ],
    )
    def sc_kernel(x_hbm, rows_hbm, cols_hbm, vals_hbm, out0_hbm, out1_hbm,
                  cols3, rows3, vals3, gat2, con2, zbuf, acc_sh,
                  si0, si1, si2, si3, sg0, sg1, ss0, ss1):
        sem_i = (si0, si1, si2, si3)
        sem_g = (sg0, sg1)
        sem_s = (ss0, ss1)
        cid = lax.axis_index("c")
        sid = lax.axis_index("s")
        wid = cid * N_SUBCORES + sid
        base = pl.multiple_of(wid * per_tile, 8)

        def idx_copies(ci, s):
            off = pl.multiple_of(base + ci * K, 8)
            return (
                pltpu.make_async_copy(cols_hbm.at[pl.ds(off, K)],
                                      cols3.at[s], sem_i[s]),
                pltpu.make_async_copy(rows_hbm.at[pl.ds(off, K)],
                                      rows3.at[s], sem_i[s]),
                pltpu.make_async_copy(vals_hbm.at[pl.ds(off, K)],
                                      vals3.at[s], sem_i[s]),
            )

        def gather_copy(s, g):
            return pltpu.make_async_copy(x_hbm.at[cols3.at[s]], gat2.at[g],
                                         sem_g[g])

        def scatter_start(s, g):
            pltpu.async_copy(con2.at[g], acc_sh.at[rows3.at[s]], sem_s[g],
                             add=True)

        def scatter_wait(s, g):
            pltpu.make_async_copy(con2.at[g], acc_sh.at[rows3.at[s]],
                                  sem_s[g]).wait()

        @pl.loop(0, ZR)
        def _(i):
            for j in range(b // 16):
                zbuf.at[pl.ds(i, 1), pl.ds(j * 16, 16)][...] = jnp.zeros(
                    (1, 16), jnp.float32)

        for c in idx_copies(0, 0):
            c.start()
        for c in idx_copies(1, 1):
            c.start()

        for c in idx_copies(0, 0):
            c.wait()
        gather_copy(0, 0).start()

        for z in range(rows_per_tile // ZR):
            pltpu.sync_copy(
                zbuf, acc_sh.at[pl.ds(sid * rows_per_tile + z * ZR, ZR)])
        plsc.subcore_barrier()

        def body(i, jmod):
            g = jmod % 2
            s = jmod % 4
            g1 = (jmod + 1) % 2
            s1 = (jmod + 1) % 4
            s2 = (jmod + 2) % 4

            gather_copy(s, g).wait()

            @pl.when(i >= 1)
            def _():
                scatter_wait((jmod + 3) % 4, g1)

            @pl.when(i + 1 < chunks)
            def _():
                for c in idx_copies(i + 1, s1):
                    c.wait()
                gather_copy(s1, g1).start()

            @pl.when(i + 2 < chunks)
            def _():
                for c in idx_copies(i + 2, s2):
                    c.start()

            gref = gat2.at[g]
            cref = con2.at[g]
            vref = vals3.at[s]

            @pl.loop(0, K, step=16)
            def _(k0):
                vv = vref[pl.ds(k0, 16)]
                for u in range(16):
                    v = vv[u]
                    for j in range(b // 16):
                        sl = (pl.ds(k0 + u, 1), pl.ds(j * 16, 16))
                        cref.at[*sl][...] = gref.at[*sl][...] * v

            scatter_start(s, g)

        @pl.loop(0, chunks, step=4)
        def _(ch):
            for jmod in range(4):
                body(ch + jmod, jmod)

        scatter_wait((chunks - 1) % 4, (chunks - 1) % 2)
        plsc.subcore_barrier()

        row0 = sid * rows_per_tile

        @pl.when(cid == 0)
        def _():
            pltpu.sync_copy(acc_sh.at[pl.ds(row0, rows_per_tile)],
                            out0_hbm.at[pl.ds(row0, rows_per_tile)])

        @pl.when(cid == 1)
        def _():
            pltpu.sync_copy(acc_sh.at[pl.ds(row0, rows_per_tile)],
                            out1_hbm.at[pl.ds(row0, rows_per_tile)])

    return sc_kernel(xT, rows, cols, vals)


def _combine_body(p0_ref, p1_ref, o_ref):
    o_ref[...] = (p0_ref[...] + p1_ref[...]).T


@functools.partial(jax.jit, static_argnames=("n", "b", "nb"))
def _combine(p0, p1, *, n, b, nb):
    return pl.pallas_call(
        _combine_body,
        grid=(n // nb,),
        in_specs=[
            pl.BlockSpec((nb, b), lambda i: (i, 0)),
            pl.BlockSpec((nb, b), lambda i: (i, 0)),
        ],
        out_specs=pl.BlockSpec((b, nb), lambda i: (0, i)),
        out_shape=jax.ShapeDtypeStruct((b, n), jnp.float32),
    )(p0, p1)


def kernel(x_affine, W_rows, W_cols, W_vals):
    b, n = x_affine.shape
    nnz = W_rows.shape[0]
    per_tile = -(-nnz // (N_TILES * 4 * K)) * 4 * K
    total = per_tile * N_TILES
    pad = total - nnz

    xT = x_affine.T
    rows = jnp.concatenate([W_rows.astype(jnp.int32),
                            jnp.zeros((pad,), jnp.int32)])
    cols = jnp.concatenate([W_cols.astype(jnp.int32),
                            jnp.zeros((pad,), jnp.int32)])
    vals = jnp.concatenate([W_vals, jnp.zeros((pad,), jnp.float32)])

    p0, p1 = _sc_spmm(xT, rows, cols, vals, n=n, b=b, per_tile=per_tile)
    return _combine(p0, p1, n=n, b=b, nb=1024)


# speedup vs baseline: 1.9638x; 1.9638x over previous
import functools

import jax
import jax.numpy as jnp
from jax import lax
from jax.experimental import pallas as pl
from jax.experimental.pallas import tpu as pltpu
from jax.experimental.pallas import tpu_sc as plsc

N_CORES = 2
N_SUBCORES = 16
N_TILES = N_CORES * N_SUBCORES
K = 128
ZR = 128


@functools.partial(jax.jit, static_argnames=("n", "b", "per_tile"))
def _sc_spmm(xT, rows, cols, vals, *, n, b, per_tile):
    chunks = per_tile // K
    rows_per_tile = n // N_SUBCORES
    mesh = plsc.VectorSubcoreMesh(core_axis_name="c", subcore_axis_name="s")

    @functools.partial(
        pl.kernel,
        out_type=(
            jax.ShapeDtypeStruct((n, b), jnp.float32),
            jax.ShapeDtypeStruct((n, b), jnp.float32),
        ),
        mesh=mesh,
        compiler_params=pltpu.CompilerParams(use_tc_tiling_on_sc=False),
        scratch_types=[
            pltpu.VMEM((3, K), jnp.int32),
            pltpu.VMEM((3, K), jnp.int32),
            pltpu.VMEM((3, K), jnp.float32),
            pltpu.VMEM((2, K, b), jnp.float32),
            pltpu.VMEM((2, K, b), jnp.float32),
            pltpu.VMEM((ZR, b), jnp.float32),
            pltpu.VMEM_SHARED((n, b), jnp.float32),
            pltpu.SemaphoreType.DMA,
            pltpu.SemaphoreType.DMA,
            pltpu.SemaphoreType.DMA,
            pltpu.SemaphoreType.DMA,
            pltpu.SemaphoreType.DMA,
            pltpu.SemaphoreType.DMA,
            pltpu.SemaphoreType.DMA,
        ],
    )
    def sc_kernel(x_hbm, rows_hbm, cols_hbm, vals_hbm, out0_hbm, out1_hbm,
                  cols3, rows3, vals3, gat2, con2, zbuf, acc_sh,
                  si0, si1, si2, sg0, sg1, ss0, ss1):
        sem_i = (si0, si1, si2)
        sem_g = (sg0, sg1)
        sem_s = (ss0, ss1)
        cid = lax.axis_index("c")
        sid = lax.axis_index("s")
        wid = cid * N_SUBCORES + sid
        base = pl.multiple_of(wid * per_tile, 8)

        def idx_copies(ci, s):
            off = pl.multiple_of(base + ci * K, 8)
            return (
                pltpu.make_async_copy(cols_hbm.at[pl.ds(off, K)],
                                      cols3.at[s], sem_i[s]),
                pltpu.make_async_copy(rows_hbm.at[pl.ds(off, K)],
                                      rows3.at[s], sem_i[s]),
                pltpu.make_async_copy(vals_hbm.at[pl.ds(off, K)],
                                      vals3.at[s], sem_i[s]),
            )

        def gather_copy(s, g):
            return pltpu.make_async_copy(x_hbm.at[cols3.at[s]], gat2.at[g],
                                         sem_g[g])

        def scatter_start(s, g):
            pltpu.async_copy(con2.at[g], acc_sh.at[rows3.at[s]], sem_s[g],
                             add=True)

        def scatter_wait(s, g):
            pltpu.make_async_copy(con2.at[g], acc_sh.at[rows3.at[s]],
                                  sem_s[g]).wait()

        @pl.loop(0, ZR)
        def _(i):
            for j in range(b // 16):
                zbuf.at[pl.ds(i, 1), pl.ds(j * 16, 16)][...] = jnp.zeros(
                    (1, 16), jnp.float32)

        for c in idx_copies(0, 0):
            c.start()
        for c in idx_copies(1, 1):
            c.start()

        for c in idx_copies(0, 0):
            c.wait()
        gather_copy(0, 0).start()

        for z in range(rows_per_tile // ZR):
            pltpu.sync_copy(
                zbuf, acc_sh.at[pl.ds(sid * rows_per_tile + z * ZR, ZR)])
        plsc.subcore_barrier()

        def body(i, jmod):
            g = jmod % 2
            s = jmod % 3
            g1 = (jmod + 1) % 2
            s1 = (jmod + 1) % 3
            s2 = (jmod + 2) % 3

            gather_copy(s, g).wait()

            @pl.when(i >= 1)
            def _():
                scatter_wait((jmod + 2) % 3, g1)

            @pl.when(i + 1 < chunks)
            def _():
                for c in idx_copies(i + 1, s1):
                    c.wait()
                gather_copy(s1, g1).start()

            @pl.when(i + 2 < chunks)
            def _():
                for c in idx_copies(i + 2, s2):
                    c.start()

            gref = gat2.at[g]
            cref = con2.at[g]
            vref = vals3.at[s]

            @pl.loop(0, K, step=16)
            def _(k0):
                vv = vref[pl.ds(k0, 16)]
                for u in range(16):
                    v = vv[u]
                    for j in range(b // 16):
                        sl = (pl.ds(k0 + u, 1), pl.ds(j * 16, 16))
                        cref.at[*sl][...] = gref.at[*sl][...] * v

            scatter_start(s, g)

        @pl.loop(0, chunks, step=6)
        def _(ch):
            for jmod in range(6):
                body(ch + jmod, jmod)

        scatter_wait((chunks - 1) % 3, (chunks - 1) % 2)
        plsc.subcore_barrier()

        row0 = sid * rows_per_tile

        @pl.when(cid == 0)
        def _():
            pltpu.sync_copy(acc_sh.at[pl.ds(row0, rows_per_tile)],
                            out0_hbm.at[pl.ds(row0, rows_per_tile)])

        @pl.when(cid == 1)
        def _():
            pltpu.sync_copy(acc_sh.at[pl.ds(row0, rows_per_tile)],
                            out1_hbm.at[pl.ds(row0, rows_per_tile)])

    return sc_kernel(xT, rows, cols, vals)


def _combine_body(p0_ref, p1_ref, o_ref):
    o_ref[...] = (p0_ref[...] + p1_ref[...]).T


@functools.partial(jax.jit, static_argnames=("n", "b", "nb"))
def _combine(p0, p1, *, n, b, nb):
    return pl.pallas_call(
        _combine_body,
        grid=(n // nb,),
        in_specs=[
            pl.BlockSpec((nb, b), lambda i: (i, 0)),
            pl.BlockSpec((nb, b), lambda i: (i, 0)),
        ],
        out_specs=pl.BlockSpec((b, nb), lambda i: (0, i)),
        out_shape=jax.ShapeDtypeStruct((b, n), jnp.float32),
    )(p0, p1)


def kernel(x_affine, W_rows, W_cols, W_vals):
    b, n = x_affine.shape
    nnz = W_rows.shape[0]
    per_tile = -(-nnz // (N_TILES * 6 * K)) * 6 * K
    total = per_tile * N_TILES
    pad = total - nnz

    xT = x_affine.T
    rows = jnp.concatenate([W_rows.astype(jnp.int32),
                            jnp.zeros((pad,), jnp.int32)])
    cols = jnp.concatenate([W_cols.astype(jnp.int32),
                            jnp.zeros((pad,), jnp.int32)])
    vals = jnp.concatenate([W_vals, jnp.zeros((pad,), jnp.float32)])

    p0, p1 = _sc_spmm(xT, rows, cols, vals, n=n, b=b, per_tile=per_tile)
    return _combine(p0, p1, n=n, b=b, nb=1024)
